# Initial kernel scaffold; baseline (speedup 1.0000x reference)
import jax
import jax.numpy as jnp
from jax.experimental import pallas as pl


def kernel(feats_r, feats_t, quantized_r, ref_index, current_ind):
    raise NotImplementedError("write your pallas kernel here")


# speedup vs baseline: 5.4271x; 5.4271x over previous
import jax
import jax.numpy as jnp
from jax.experimental import pallas as pl
from jax.experimental.pallas import tpu as pltpu

D = 4
R = 6
C = 16
P = 2 * R + 1
N = P * P
DIL = 15


def _colorizer_body(maskadd_ref, ft_ref, frp_ref, labp_ref, out_ref,
                    corr_ref, acc_ref, z_ref, m_ref):
    nref = frp_ref.shape[0]
    hh, ww = ft_ref.shape[1], ft_ref.shape[2]

    m_ref[0] = jnp.full((hh, ww), -jnp.inf, dtype=jnp.float32)

    def corr_body(t, _):
        i = t // P
        dy = t % P
        slab = frp_ref[i, :, pl.ds(dy, hh), :]
        madd = maskadd_ref[i, 0, 0]
        mloc = m_ref[0]
        for dx in range(P):
            cslab = jnp.sum(ft_ref[:, :, :] * slab[:, :, dx:dx + ww],
                            axis=0) + madd
            corr_ref[t * P + dx] = cslab
            mloc = jnp.maximum(mloc, cslab)
        m_ref[0] = mloc
        return 0

    jax.lax.fori_loop(0, nref * P, corr_body, 0, unroll=False)

    acc_ref[:, :, :] = jnp.zeros((C, hh, ww), dtype=jnp.float32)
    z_ref[0] = jnp.zeros((hh, ww), dtype=jnp.float32)

    def acc_body(t, _):
        i = t // P
        dy = t % P
        lab_slab = labp_ref[i, pl.ds(dy, hh), :]
        m = m_ref[0]
        z = z_ref[0]
        for dx in range(P):
            e = jnp.exp(corr_ref[t * P + dx] - m)
            z = z + e
            lab = lab_slab[:, dx:dx + ww]
            for c in range(C):
                acc_ref[c] += jnp.where(lab == c, e, 0.0)
        z_ref[0] = z
        return 0

    jax.lax.fori_loop(0, nref * P, acc_body, 0, unroll=False)

    inv = 1.0 / z_ref[0]
    out_ref[0] = acc_ref[:, :, :] * inv[None, :, :]


def kernel(feats_r, feats_t, quantized_r, ref_index, current_ind):
    nref, b, cf, hh, ww = feats_r.shape
    ft = feats_t[0]
    fr = feats_r[:, 0]
    frp = jnp.pad(fr, ((0, 0), (0, 0), (R, R), (R, R)))
    q = quantized_r[:, 0, 0, ::D, ::D].astype(jnp.int32)
    labp = jnp.pad(q, ((0, 0), (R, R), (R, R)), constant_values=-1)

    nsearch = jnp.sum((current_ind - ref_index) > DIL)
    keep = jnp.arange(nref) >= nsearch
    maskadd = jnp.where(keep, 0.0, -jnp.inf).astype(jnp.float32)
    maskadd = maskadd.reshape(nref, 1, 1)

    out = pl.pallas_call(
        _colorizer_body,
        out_shape=jax.ShapeDtypeStruct((1, C, hh, ww), jnp.float32),
        scratch_shapes=[
            pltpu.VMEM((nref * N, hh, ww), jnp.float32),
            pltpu.VMEM((C, hh, ww), jnp.float32),
            pltpu.VMEM((1, hh, ww), jnp.float32),
            pltpu.VMEM((1, hh, ww), jnp.float32),
        ],
    )(maskadd, ft, frp, labp)
    return out


# speedup vs baseline: 14.7662x; 2.7208x over previous
import jax
import jax.numpy as jnp
from jax.experimental import pallas as pl
from jax.experimental.pallas import tpu as pltpu

D = 4
R = 6
C = 16
P = 2 * R + 1
N = P * P
DIL = 15
HH = 56
WW = 56
CF = 64
NREF = 2
W2 = 2 * WW
CH = CF // 2


def _colorizer_body(maskadd_ref, ft_ref, frp_ref, labp_ref, out_ref,
                    ftp_ref, frxp_ref, labm_ref, corr_ref, acc_ref,
                    z_ref, m_ref):
    ftp_ref[...] = jnp.concatenate(
        (ft_ref[:CH], ft_ref[CH:CF]), axis=2)
    for dx in range(P):
        src = frp_ref[:, :, :, dx:dx + WW]
        frxp_ref[dx] = jnp.concatenate(
            (src[:, :CH], src[:, CH:CF]), axis=3)
        sh = labp_ref[:, :, dx:dx + WW]
        for c in range(C):
            m0 = (sh[0] == c).astype(jnp.float32)
            m1 = (sh[1] == c).astype(jnp.float32)
            labm_ref[dx, c] = jnp.concatenate((m0, m1), axis=1)

    m_ref[0] = jnp.full((HH, W2), -jnp.inf, dtype=jnp.float32)

    def corr_body(dy, _):
        mloc = m_ref[0]
        for dx in range(P):
            halves = []
            for i in range(NREF):
                res = jnp.sum(
                    ftp_ref[:, :, :] *
                    frxp_ref[dx, i, :, pl.ds(dy, HH), :], axis=0)
                halves.append(res[:, :WW] + res[:, WW:] +
                              maskadd_ref[i, 0, 0])
            packed = jnp.concatenate(halves, axis=1)
            corr_ref[dy * P + dx] = packed
            mloc = jnp.maximum(mloc, packed)
        m_ref[0] = mloc
        return 0

    jax.lax.fori_loop(0, P, corr_body, 0, unroll=False)

    mm = jnp.maximum(m_ref[0][:, :WW], m_ref[0][:, WW:])
    m_ref[0] = jnp.concatenate((mm, mm), axis=1)

    acc_ref[...] = jnp.zeros((C, HH, W2), dtype=jnp.float32)
    z_ref[0] = jnp.zeros((HH, W2), dtype=jnp.float32)

    def acc_body(dy, _):
        m = m_ref[0]
        es = [jnp.exp(corr_ref[dy * P + dx] - m) for dx in range(P)]
        zs = es[0]
        for dx in range(1, P):
            zs = zs + es[dx]
        z_ref[0] += zs
        for c in range(C):
            a = es[0] * labm_ref[0, c, pl.ds(dy, HH), :]
            for dx in range(1, P):
                a = a + es[dx] * labm_ref[dx, c, pl.ds(dy, HH), :]
            acc_ref[c] += a
        return 0

    jax.lax.fori_loop(0, P, acc_body, 0, unroll=False)

    z = z_ref[0]
    inv = 1.0 / (z[:, :WW] + z[:, WW:])
    for c in range(C):
        a = acc_ref[c]
        out_ref[0, c] = (a[:, :WW] + a[:, WW:]) * inv


def kernel(feats_r, feats_t, quantized_r, ref_index, current_ind):
    nref, b, cf, hh, ww = feats_r.shape
    ft = feats_t[0]
    fr = feats_r[:, 0]
    frp = jnp.pad(fr, ((0, 0), (0, 0), (R, R), (R, R)))
    q = quantized_r[:, 0, 0, ::D, ::D].astype(jnp.int32)
    labp = jnp.pad(q, ((0, 0), (R, R), (R, R)), constant_values=-1)

    nsearch = jnp.sum((current_ind - ref_index) > DIL)
    keep = jnp.arange(nref) >= nsearch
    maskadd = jnp.where(keep, 0.0, -jnp.inf).astype(jnp.float32)
    maskadd = maskadd.reshape(nref, 1, 1)

    out = pl.pallas_call(
        _colorizer_body,
        out_shape=jax.ShapeDtypeStruct((1, C, hh, ww), jnp.float32),
        scratch_shapes=[
            pltpu.VMEM((CH, HH, W2), jnp.float32),
            pltpu.VMEM((P, NREF, CH, HH + 2 * R, W2), jnp.float32),
            pltpu.VMEM((P, C, HH + 2 * R, W2), jnp.float32),
            pltpu.VMEM((N, HH, W2), jnp.float32),
            pltpu.VMEM((C, HH, W2), jnp.float32),
            pltpu.VMEM((1, HH, W2), jnp.float32),
            pltpu.VMEM((1, HH, W2), jnp.float32),
        ],
    )(maskadd, ft, frp, labp)
    return out
